# baseline (speedup 1.0000x reference)
import functools

import jax
import jax.numpy as jnp
from jax import lax
from jax.experimental import pallas as pl
from jax.experimental.pallas import tpu as pltpu
from jax.experimental.pallas import tpu_sc as plsc

B = 4096
L = 200
D = 32
N_TOTAL = B * L
V = 1000000

GRP = 8192
BLKR = GRP // 4
NGRP = -(-V // GRP)
VP = NGRP * BLKR * 4

NC = 2
NS = 16
NW = NC * NS
PER_W = N_TOTAL // NW
CHUNK = 128
N_CHUNK = PER_W // CHUNK


def _tc_transform(table_t, wt, bias):
    def body(x_ref, w_ref, b_ref, o_ref):
        w = w_ref[...]
        bb = b_ref[...]
        x = x_ref[...]
        cols = []
        for k in range(4):
            acc = lax.dot_general(
                x[:, k * BLKR:(k + 1) * BLKR], w, (((0,), (0,)), ((), ())),
                preferred_element_type=jnp.float32)
            cols.append(jnp.tanh(acc + bb))
        o_ref[...] = jnp.concatenate(cols, axis=1)

    return pl.pallas_call(
        body,
        grid=(NGRP,),
        in_specs=[
            pl.BlockSpec((32, GRP), lambda i: (0, i)),
            pl.BlockSpec((32, 32), lambda i: (0, 0)),
            pl.BlockSpec((1, 32), lambda i: (0, 0)),
        ],
        out_specs=pl.BlockSpec((BLKR, 128), lambda i: (i, 0)),
        out_shape=jax.ShapeDtypeStruct((VP // 4, 128), jnp.float32),
    )(table_t, wt, bias)


def _sc_gather_t(table, idx3):
    mesh = plsc.VectorSubcoreMesh(core_axis_name="c", subcore_axis_name="s")

    @functools.partial(
        pl.kernel,
        out_type=jax.ShapeDtypeStruct((L, D, B), jnp.float32),
        mesh=mesh,
        scratch_types=[
            pltpu.VMEM((L, CHUNK), jnp.int32),
            pltpu.VMEM((CHUNK, D), jnp.float32),
            pltpu.VMEM((CHUNK, D), jnp.float32),
            pltpu.VMEM((D, CHUNK), jnp.float32),
            pltpu.VMEM((D, CHUNK), jnp.float32),
            pltpu.SemaphoreType.DMA,
            pltpu.SemaphoreType.DMA,
            pltpu.SemaphoreType.DMA,
            pltpu.SemaphoreType.DMA,
        ],
        compiler_params=pltpu.CompilerParams(
            use_tc_tiling_on_sc=False, needs_layout_passes=False),
    )
    def k(table_hbm, idx_hbm, out_hbm, idx_v, rows_a, rows_b, tr_a, tr_b,
          sga, sgb, soa, sob):
        wid = lax.axis_index("s") * NC + lax.axis_index("c")
        b0 = wid * CHUNK
        pltpu.sync_copy(idx_hbm.at[wid], idx_v)
        lane = lax.iota(jnp.int32, 16)

        def transpose(rows_v, tr_v):
            for j in range(D):
                for c0 in range(0, CHUNK, 16):
                    ridx = lane + c0
                    cidx = jnp.full((16,), j, jnp.int32)
                    tr_v[j, pl.ds(c0, 16)] = plsc.load_gather(
                        rows_v, [ridx, cidx])

        pltpu.async_copy(table_hbm.at[idx_v.at[0]], rows_a, sga)

        @pl.loop(0, L // 2)
        def _(p):
            l = p * 2
            pltpu.async_copy(table_hbm.at[idx_v.at[l + 1]], rows_b, sgb)
            pltpu.make_async_copy(table_hbm.at[idx_v.at[l]], rows_a, sga).wait()

            @pl.when(l >= 2)
            def _():
                pltpu.make_async_copy(
                    tr_a, out_hbm.at[l - 2, :, pl.ds(b0, CHUNK)], soa).wait()

            transpose(rows_a, tr_a)
            pltpu.async_copy(tr_a, out_hbm.at[l, :, pl.ds(b0, CHUNK)], soa)

            @pl.when(l + 2 < L)
            def _():
                pltpu.async_copy(table_hbm.at[idx_v.at[l + 2]], rows_a, sga)

            pltpu.make_async_copy(table_hbm.at[idx_v.at[l + 1]], rows_b, sgb).wait()

            @pl.when(l >= 2)
            def _():
                pltpu.make_async_copy(
                    tr_b, out_hbm.at[l - 1, :, pl.ds(b0, CHUNK)], sob).wait()

            transpose(rows_b, tr_b)
            pltpu.async_copy(tr_b, out_hbm.at[l + 1, :, pl.ds(b0, CHUNK)], sob)

        pltpu.make_async_copy(
            tr_a, out_hbm.at[L - 2, :, pl.ds(b0, CHUNK)], soa).wait()
        pltpu.make_async_copy(
            tr_b, out_hbm.at[L - 1, :, pl.ds(b0, CHUNK)], sob).wait()

    return k(table, idx3)


def kernel(x, table, W, b):
    t128 = _tc_transform(table.T, W.T, b.reshape(1, D))

    xi = x.astype(jnp.int32)
    g = xi >> 13
    w = xi & (GRP - 1)
    k = w >> 11
    r = w & (BLKR - 1)
    xj = (((g << 11) + r) << 2) + k

    idx3 = xj.T.reshape(L, NW, CHUNK).transpose(1, 0, 2)
    out = _sc_gather_t(t128.reshape(VP, D), idx3)
    return out.transpose(2, 0, 1)


# speedup vs baseline: 2.1071x; 2.1071x over previous
import functools

import jax
import jax.numpy as jnp
from jax import lax
from jax.experimental import pallas as pl
from jax.experimental.pallas import tpu as pltpu
from jax.experimental.pallas import tpu_sc as plsc

B = 4096
L = 200
D = 32
N_TOTAL = B * L
V = 1000000

GRP = 8192
BLKR = GRP // 4
NGRP = -(-V // GRP)
VP = NGRP * BLKR * 4

NC = 2
NS = 16
NW = NC * NS
PER_W = N_TOTAL // NW
CHUNK = 128
N_CHUNK = PER_W // CHUNK


def _tc_transform(table_t, wt, bias):
    def body(x_ref, w_ref, b_ref, o_ref):
        w = w_ref[...]
        bb = b_ref[...]
        x = x_ref[...]
        rows = []
        for k in range(4):
            acc = lax.dot_general(
                w, x[:, k * BLKR:(k + 1) * BLKR], (((0,), (0,)), ((), ())),
                preferred_element_type=jnp.float32)
            rows.append(jnp.tanh(acc + bb))
        o_ref[...] = jnp.concatenate(rows, axis=0).T

    return pl.pallas_call(
        body,
        grid=(NGRP,),
        in_specs=[
            pl.BlockSpec((32, GRP), lambda i: (0, i)),
            pl.BlockSpec((32, 32), lambda i: (0, 0)),
            pl.BlockSpec((32, 1), lambda i: (0, 0)),
        ],
        out_specs=pl.BlockSpec((BLKR, 128), lambda i: (i, 0)),
        out_shape=jax.ShapeDtypeStruct((VP // 4, 128), jnp.float32),
        compiler_params=pltpu.CompilerParams(
            fuse_transposed_lhs_in_matmul=True),
    )(table_t, wt, bias)


def _sc_gather(table, idx3):
    mesh = plsc.VectorSubcoreMesh(core_axis_name="c", subcore_axis_name="s")

    @functools.partial(
        pl.kernel,
        out_type=jax.ShapeDtypeStruct((N_TOTAL, D), jnp.float32),
        mesh=mesh,
        scratch_types=[
            pltpu.VMEM((N_CHUNK, CHUNK), jnp.int32),
            pltpu.VMEM((CHUNK, D), jnp.float32),
            pltpu.VMEM((CHUNK, D), jnp.float32),
            pltpu.SemaphoreType.DMA,
            pltpu.SemaphoreType.DMA,
        ],
        compiler_params=pltpu.CompilerParams(use_tc_tiling_on_sc=False),
    )
    def k(table_hbm, idx_hbm, out_hbm, idx_v, rows_a, rows_b, sem_a, sem_b):
        wid = lax.axis_index("s") * NC + lax.axis_index("c")
        base = wid * PER_W
        pltpu.sync_copy(idx_hbm.at[wid], idx_v)

        pltpu.async_copy(table_hbm.at[idx_v.at[0]], rows_a, sem_a)

        @pl.loop(0, N_CHUNK // 2)
        def _(p):
            j = p * 2
            pltpu.async_copy(table_hbm.at[idx_v.at[j + 1]], rows_b, sem_b)
            pltpu.make_async_copy(table_hbm.at[idx_v.at[j]], rows_a, sem_a).wait()
            pltpu.sync_copy(rows_a, out_hbm.at[pl.ds(base + j * CHUNK, CHUNK)])

            @pl.when(j + 2 < N_CHUNK)
            def _():
                pltpu.async_copy(table_hbm.at[idx_v.at[j + 2]], rows_a, sem_a)

            pltpu.make_async_copy(table_hbm.at[idx_v.at[j + 1]], rows_b, sem_b).wait()
            pltpu.sync_copy(rows_b, out_hbm.at[pl.ds(base + (j + 1) * CHUNK, CHUNK)])

    return k(table, idx3)


TB = 512
TF = 640


def _tc_transpose(a):
    def body(x_ref, o_ref):
        o_ref[...] = x_ref[...].T

    return pl.pallas_call(
        body,
        grid=(B // TB, L * D // TF),
        in_specs=[pl.BlockSpec((TB, TF), lambda i, j: (i, j))],
        out_specs=pl.BlockSpec((TF, TB), lambda i, j: (j, i)),
        out_shape=jax.ShapeDtypeStruct((L * D, B), jnp.float32),
    )(a)


def kernel(x, table, W, b):
    t128 = _tc_transform(table.T, W.T, b.reshape(D, 1))

    xi = x.astype(jnp.int32)
    g = xi >> 13
    w = xi & (GRP - 1)
    k = w >> 11
    r = w & (BLKR - 1)
    xj = (((g << 11) + r) << 2) + k

    idx3 = xj.reshape(NW, N_CHUNK, CHUNK)
    out = _sc_gather(t128.reshape(VP, D), idx3)

    ot = _tc_transpose(out.reshape(B, L * D))
    return ot.reshape(L, D, B).transpose(2, 0, 1)


# speedup vs baseline: 2.4453x; 1.1605x over previous
import functools

import jax
import jax.numpy as jnp
from jax import lax
from jax.experimental import pallas as pl
from jax.experimental.pallas import tpu as pltpu
from jax.experimental.pallas import tpu_sc as plsc

B = 4096
L = 200
D = 32
N_TOTAL = B * L
V = 1000000

GRP = 16384
BLKR = GRP // 4
GRP_SH = GRP.bit_length() - 1
BLK_SH = BLKR.bit_length() - 1
NGRP = -(-V // GRP)
VP = NGRP * BLKR * 4

NC = 2
NS = 16
NW = NC * NS
PER_W = N_TOTAL // NW
CHUNK = 128
N_CHUNK = PER_W // CHUNK


def _tc_transform(table_t, wt, bias):
    def body(x_ref, w_ref, b_ref, o_ref):
        w = w_ref[...]
        bb = b_ref[...]
        x = x_ref[...]
        rows = []
        for k in range(4):
            acc = lax.dot_general(
                w, x[:, k * BLKR:(k + 1) * BLKR], (((0,), (0,)), ((), ())),
                preferred_element_type=jnp.float32)
            rows.append(jnp.tanh(acc + bb))
        o_ref[...] = jnp.concatenate(rows, axis=0).T

    return pl.pallas_call(
        body,
        grid=(NGRP,),
        in_specs=[
            pl.BlockSpec((32, GRP), lambda i: (0, i)),
            pl.BlockSpec((32, 32), lambda i: (0, 0)),
            pl.BlockSpec((32, 1), lambda i: (0, 0)),
        ],
        out_specs=pl.BlockSpec((BLKR, 128), lambda i: (i, 0)),
        out_shape=jax.ShapeDtypeStruct((VP // 4, 128), jnp.float32),
        compiler_params=pltpu.CompilerParams(
            fuse_transposed_lhs_in_matmul=True),
    )(table_t, wt, bias)


def _sc_gather(table, idx3):
    mesh = plsc.VectorSubcoreMesh(core_axis_name="c", subcore_axis_name="s")

    @functools.partial(
        pl.kernel,
        out_type=jax.ShapeDtypeStruct((N_TOTAL, D), jnp.float32),
        mesh=mesh,
        scratch_types=[
            pltpu.VMEM((N_CHUNK, CHUNK), jnp.int32),
            pltpu.VMEM((CHUNK, D), jnp.float32),
            pltpu.VMEM((CHUNK, D), jnp.float32),
            pltpu.SemaphoreType.DMA,
            pltpu.SemaphoreType.DMA,
        ],
        compiler_params=pltpu.CompilerParams(use_tc_tiling_on_sc=False),
    )
    def k(table_hbm, idx_hbm, out_hbm, idx_v, rows_a, rows_b, sem_a, sem_b):
        wid = lax.axis_index("s") * NC + lax.axis_index("c")
        base = wid * PER_W
        pltpu.sync_copy(idx_hbm.at[wid], idx_v)

        pltpu.async_copy(table_hbm.at[idx_v.at[0]], rows_a, sem_a)

        @pl.loop(0, N_CHUNK // 2)
        def _(p):
            j = p * 2
            pltpu.async_copy(table_hbm.at[idx_v.at[j + 1]], rows_b, sem_b)
            pltpu.make_async_copy(table_hbm.at[idx_v.at[j]], rows_a, sem_a).wait()
            pltpu.sync_copy(rows_a, out_hbm.at[pl.ds(base + j * CHUNK, CHUNK)])

            @pl.when(j + 2 < N_CHUNK)
            def _():
                pltpu.async_copy(table_hbm.at[idx_v.at[j + 2]], rows_a, sem_a)

            pltpu.make_async_copy(table_hbm.at[idx_v.at[j + 1]], rows_b, sem_b).wait()
            pltpu.sync_copy(rows_b, out_hbm.at[pl.ds(base + (j + 1) * CHUNK, CHUNK)])

    return k(table, idx3)


TB = 1024
TF = 1280


def _tc_transpose(a):
    def body(x_ref, o_ref):
        o_ref[...] = x_ref[...].T

    return pl.pallas_call(
        body,
        grid=(B // TB, L * D // TF),
        in_specs=[pl.BlockSpec((TB, TF), lambda i, j: (i, j))],
        out_specs=pl.BlockSpec((TF, TB), lambda i, j: (j, i)),
        out_shape=jax.ShapeDtypeStruct((L * D, B), jnp.float32),
    )(a)


def kernel(x, table, W, b):
    t128 = _tc_transform(table.T, W.T, b.reshape(D, 1))

    xi = x.astype(jnp.int32)
    g = xi >> GRP_SH
    w = xi & (GRP - 1)
    k = w >> BLK_SH
    r = w & (BLKR - 1)
    xj = (((g << BLK_SH) + r) << 2) + k

    idx3 = xj.reshape(NW, N_CHUNK, CHUNK)
    out = _sc_gather(t128.reshape(VP, D), idx3)

    ot = _tc_transpose(out.reshape(B, L * D))
    return ot.reshape(L, D, B).transpose(2, 0, 1)


# speedup vs baseline: 2.4510x; 1.0023x over previous
import functools

import jax
import jax.numpy as jnp
from jax import lax
from jax.experimental import pallas as pl
from jax.experimental.pallas import tpu as pltpu
from jax.experimental.pallas import tpu_sc as plsc

B = 4096
L = 200
D = 32
N_TOTAL = B * L
V = 1000000

GRP = 16384
BLKR = GRP // 4
GRP_SH = GRP.bit_length() - 1
BLK_SH = BLKR.bit_length() - 1
NGRP = -(-V // GRP)
VP = NGRP * BLKR * 4

NC = 2
NS = 16
NW = NC * NS
PER_W = N_TOTAL // NW
CHUNK = 128
N_CHUNK = PER_W // CHUNK


def _tc_transform(table_t, wt, bias):
    def body(x_ref, w_ref, b_ref, o_ref):
        w = w_ref[...]
        bb = b_ref[...]
        x = x_ref[...]
        rows = []
        for k in range(4):
            acc = lax.dot_general(
                w, x[:, k * BLKR:(k + 1) * BLKR], (((0,), (0,)), ((), ())),
                preferred_element_type=jnp.float32)
            rows.append(jnp.tanh(acc + bb))
        o_ref[...] = jnp.concatenate(rows, axis=0).T

    return pl.pallas_call(
        body,
        grid=(NGRP,),
        in_specs=[
            pl.BlockSpec((32, GRP), lambda i: (0, i)),
            pl.BlockSpec((32, 32), lambda i: (0, 0)),
            pl.BlockSpec((32, 1), lambda i: (0, 0)),
        ],
        out_specs=pl.BlockSpec((BLKR, 128), lambda i: (i, 0)),
        out_shape=jax.ShapeDtypeStruct((VP // 4, 128), jnp.float32),
        compiler_params=pltpu.CompilerParams(
            fuse_transposed_lhs_in_matmul=True),
    )(table_t, wt, bias)


def _sc_gather(table, idx3):
    mesh = plsc.VectorSubcoreMesh(core_axis_name="c", subcore_axis_name="s")

    @functools.partial(
        pl.kernel,
        out_type=jax.ShapeDtypeStruct((N_TOTAL, D), jnp.float32),
        mesh=mesh,
        scratch_types=[
            pltpu.VMEM((N_CHUNK, CHUNK), jnp.int32),
            pltpu.VMEM((CHUNK, D), jnp.float32),
            pltpu.VMEM((CHUNK, D), jnp.float32),
            pltpu.SemaphoreType.DMA,
            pltpu.SemaphoreType.DMA,
        ],
        compiler_params=pltpu.CompilerParams(use_tc_tiling_on_sc=False),
    )
    def k(table_hbm, idx_hbm, out_hbm, idx_v, rows_a, rows_b, sem_a, sem_b):
        wid = lax.axis_index("s") * NC + lax.axis_index("c")
        base = wid * PER_W
        pltpu.sync_copy(idx_hbm.at[wid], idx_v)

        pltpu.async_copy(table_hbm.at[idx_v.at[0]], rows_a, sem_a)

        @pl.loop(0, N_CHUNK // 2)
        def _(p):
            j = p * 2
            pltpu.async_copy(table_hbm.at[idx_v.at[j + 1]], rows_b, sem_b)
            pltpu.make_async_copy(table_hbm.at[idx_v.at[j]], rows_a, sem_a).wait()
            pltpu.sync_copy(rows_a, out_hbm.at[pl.ds(base + j * CHUNK, CHUNK)])

            @pl.when(j + 2 < N_CHUNK)
            def _():
                pltpu.async_copy(table_hbm.at[idx_v.at[j + 2]], rows_a, sem_a)

            pltpu.make_async_copy(table_hbm.at[idx_v.at[j + 1]], rows_b, sem_b).wait()
            pltpu.sync_copy(rows_b, out_hbm.at[pl.ds(base + (j + 1) * CHUNK, CHUNK)])

    return k(table, idx3)


TB = 2048
TF = 1280


def _tc_transpose(a):
    def body(x_ref, o_ref):
        o_ref[...] = x_ref[...].T

    return pl.pallas_call(
        body,
        grid=(B // TB, L * D // TF),
        in_specs=[pl.BlockSpec((TB, TF), lambda i, j: (i, j))],
        out_specs=pl.BlockSpec((TF, TB), lambda i, j: (j, i)),
        out_shape=jax.ShapeDtypeStruct((L * D, B), jnp.float32),
    )(a)


def kernel(x, table, W, b):
    t128 = _tc_transform(table.T, W.T, b.reshape(D, 1))

    xi = x.astype(jnp.int32)
    g = xi >> GRP_SH
    w = xi & (GRP - 1)
    k = w >> BLK_SH
    r = w & (BLKR - 1)
    xj = (((g << BLK_SH) + r) << 2) + k

    idx3 = xj.reshape(NW, N_CHUNK, CHUNK)
    out = _sc_gather(t128.reshape(VP, D), idx3)

    ot = _tc_transpose(out.reshape(B, L * D))
    return ot.reshape(L, D, B).transpose(2, 0, 1)
